# baseline (speedup 1.0000x reference)
import functools

import jax
import jax.numpy as jnp
from jax import lax
from jax.experimental import pallas as pl
from jax.experimental.pallas import tpu as pltpu
from jax.experimental.pallas import tpu_sc as plsc

N = 10000
E = 160000
DIN = 256
H = 512
NQ = 4
QW = 128
NT = 16
CHUNK = 128
CPT = 79
CE = 64
NG = 2
GC = 79
EPAD = NT * CPT * CHUNK
NSL = 10240
ZPT = NSL // NT
DW = 16
MB = 1000

_SC_MESH = plsc.VectorSubcoreMesh(core_axis_name="c", subcore_axis_name="s")


def _zero_vec():
    return jnp.zeros((16,), jnp.float32)


@functools.partial(
    pl.kernel,
    out_type=jax.ShapeDtypeStruct((NSL, QW), jnp.float32),
    mesh=_SC_MESH,
    scratch_types=[
        pltpu.VMEM((CPT, CHUNK), jnp.int32),
        pltpu.VMEM((CHUNK, QW), jnp.float32),
        pltpu.VMEM((CHUNK, QW), jnp.float32),
        pltpu.VMEM_SHARED((NSL, QW), jnp.float32),
    ],
)
def _hist_dinv(dst_hbm, dinv_hbm, dbuf, ones_b, sbuf, hist_sh):
    c = lax.axis_index("c")
    s = lax.axis_index("s")
    one = jnp.ones((16,), jnp.float32)

    def fo(i, carry):
        for l in range(QW // 16):
            ones_b[i, pl.ds(l * 16, 16)] = one
            sbuf[i, pl.ds(l * 16, 16)] = _zero_vec()
        return carry

    lax.fori_loop(0, CHUNK, fo, 0)
    for k in range(ZPT // CHUNK):
        pltpu.sync_copy(sbuf, hist_sh.at[pl.ds(s * ZPT + k * CHUNK, CHUNK)])
    plsc.subcore_barrier()

    pltpu.sync_copy(dst_hbm.at[s], dbuf)

    def acc(i, carry):
        pltpu.sync_copy(ones_b, hist_sh.at[dbuf.at[i]], add=True)
        return carry

    lax.fori_loop(0, CPT, acc, 0)
    plsc.subcore_barrier()

    for k in range(ZPT // CHUNK):
        pltpu.sync_copy(hist_sh.at[pl.ds(s * ZPT + k * CHUNK, CHUNK)], sbuf)

        def dv(r, carry):
            v = sbuf[r, pl.ds(0, 16)] + 1.0
            ii = lax.bitcast_convert_type(v, jnp.int32)
            ii = 0x5F3759DF - lax.shift_right_arithmetic(ii, 1)
            y = lax.bitcast_convert_type(ii, jnp.float32)
            y = y * (1.5 - 0.5 * v * y * y)
            y = y * (1.5 - 0.5 * v * y * y)
            y = y * (1.5 - 0.5 * v * y * y)
            sbuf[r, pl.ds(0, 16)] = y
            return carry

        lax.fori_loop(0, CHUNK, dv, 0)

        @pl.when(c == 0)
        def _():
            pltpu.sync_copy(sbuf, dinv_hbm.at[pl.ds(s * ZPT + k * CHUNK, CHUNK)])


@functools.partial(
    pl.kernel,
    out_type=jax.ShapeDtypeStruct((NQ * N, QW), jnp.float32),
    mesh=_SC_MESH,
    scratch_types=[
        pltpu.VMEM((GC, CE), jnp.int32),
        pltpu.VMEM((GC, CE), jnp.int32),
        pltpu.VMEM((CE, QW), jnp.float32),
        pltpu.VMEM((CE, QW), jnp.float32),
        pltpu.VMEM_SHARED((NSL, QW), jnp.float32),
        pltpu.SemaphoreType.DMA,
        pltpu.SemaphoreType.DMA,
    ],
)
def _agg(src_hbm, dst_hbm, hws_hbm, agg_hbm, isrc, idst, rows0, rows1,
         slab, g0, g1):
    c = lax.axis_index("c")
    s = lax.axis_index("s")
    rbase = s * ZPT

    for sub in range(2):
        q = c * 2 + sub

        def zr(i, carry):
            for l in range(QW // 16):
                rows0[i, pl.ds(l * 16, 16)] = _zero_vec()
            return carry

        lax.fori_loop(0, CE, zr, 0)
        for k in range(ZPT // CE):
            pltpu.sync_copy(rows0, slab.at[pl.ds(rbase + k * CE, CE)])
        plsc.subcore_barrier()

        for g in range(NG):
            pltpu.sync_copy(src_hbm.at[q, s, g], isrc)
            pltpu.sync_copy(dst_hbm.at[s, g], idst)

            pltpu.async_copy(hws_hbm.at[isrc.at[0]], rows0, g0)

            def pair(j, carry):
                i0 = 2 * j
                pltpu.make_async_copy(hws_hbm.at[isrc.at[i0]], rows0, g0).wait()
                pltpu.async_copy(hws_hbm.at[isrc.at[i0 + 1]], rows1, g1)
                pltpu.sync_copy(rows0, slab.at[idst.at[i0]], add=True)
                pltpu.make_async_copy(
                    hws_hbm.at[isrc.at[i0 + 1]], rows1, g1).wait()
                pltpu.async_copy(hws_hbm.at[isrc.at[i0 + 2]], rows0, g0)
                pltpu.sync_copy(rows1, slab.at[idst.at[i0 + 1]], add=True)
                return carry

            lax.fori_loop(0, GC // 2, pair, 0)
            pltpu.make_async_copy(hws_hbm.at[isrc.at[GC - 1]], rows0, g0).wait()
            pltpu.sync_copy(rows0, slab.at[idst.at[GC - 1]], add=True)
        plsc.subcore_barrier()

        @pl.when(s < NT - 1)
        def _():
            pltpu.sync_copy(
                slab.at[pl.ds(rbase, ZPT)],
                agg_hbm.at[pl.ds(q * N + rbase, ZPT)],
            )

        @pl.when(s == NT - 1)
        def _():
            pltpu.sync_copy(
                slab.at[pl.ds(rbase, N - (NT - 1) * ZPT)],
                agg_hbm.at[pl.ds(q * N + rbase, N - (NT - 1) * ZPT)],
            )


def _front_body(x_ref, win_ref, bin_ref, w1_ref, dinv_ref, out_ref):
    h0 = jnp.maximum(
        jnp.dot(x_ref[...], win_ref[...], preferred_element_type=jnp.float32)
        + bin_ref[...],
        0.0,
    )
    hw = jnp.dot(h0, w1_ref[...], preferred_element_type=jnp.float32)
    hws = hw * dinv_ref[...]
    for qq in range(NQ):
        out_ref[qq] = hws[:, qq * QW:(qq + 1) * QW]


_tc_front = pl.pallas_call(
    _front_body,
    grid=(N // MB,),
    in_specs=[
        pl.BlockSpec((MB, DIN), lambda i: (i, 0)),
        pl.BlockSpec((DIN, H), lambda i: (0, 0)),
        pl.BlockSpec((1, H), lambda i: (0, 0)),
        pl.BlockSpec((H, H), lambda i: (0, 0)),
        pl.BlockSpec((MB, 1), lambda i: (i, 0)),
    ],
    out_specs=pl.BlockSpec((NQ, MB, QW), lambda i: (0, i, 0)),
    out_shape=jax.ShapeDtypeStruct((NQ, N, QW), jnp.float32),
)


def _mid_body(agg_ref, hws_ref, dinv_ref, b1_ref, w2_ref, out_ref):
    qi = pl.program_id(2)
    h = jnp.maximum(
        (agg_ref[0] + hws_ref[0]) * dinv_ref[...] + b1_ref[0], 0.0
    )
    p = jnp.dot(h, w2_ref[...], preferred_element_type=jnp.float32)

    @pl.when(qi == 0)
    def _():
        out_ref[0] = p

    @pl.when(qi > 0)
    def _():
        out_ref[0] = out_ref[0] + p

    @pl.when(qi == NQ - 1)
    def _():
        out_ref[0] = out_ref[0] * dinv_ref[...]


_tc_mid = pl.pallas_call(
    _mid_body,
    grid=(N // MB, NQ, NQ),
    in_specs=[
        pl.BlockSpec((1, MB, QW), lambda i, qo, qi: (qi, i, 0)),
        pl.BlockSpec((1, MB, QW), lambda i, qo, qi: (qi, i, 0)),
        pl.BlockSpec((MB, 1), lambda i, qo, qi: (i, 0)),
        pl.BlockSpec((1, 1, QW), lambda i, qo, qi: (qi, 0, 0)),
        pl.BlockSpec((QW, QW), lambda i, qo, qi: (qi, qo)),
    ],
    out_specs=pl.BlockSpec((1, MB, QW), lambda i, qo, qi: (qo, i, 0)),
    out_shape=jax.ShapeDtypeStruct((NQ, N, QW), jnp.float32),
)


def _final_body(agg_ref, hws_ref, dinv_ref, b2_ref, wout_ref, bout_ref,
                alpha_ref, rs_ref, out_ref):
    a = 1.0 / (1.0 + jnp.exp(-alpha_ref[0, 0]))
    dinv = dinv_ref[...]
    acc = jnp.zeros((MB, 1), jnp.float32)
    for qq in range(NQ):
        h2 = jnp.maximum(
            (agg_ref[qq] + hws_ref[qq]) * dinv + b2_ref[qq:qq + 1, :], 0.0
        )
        acc = acc + jnp.sum(h2 * wout_ref[qq:qq + 1, :], axis=1, keepdims=True)
    gnn = acc + bout_ref[0, 0]
    out_ref[...] = a * rs_ref[...] + (1.0 - a) * gnn


_tc_final = pl.pallas_call(
    _final_body,
    grid=(N // MB,),
    in_specs=[
        pl.BlockSpec((NQ, MB, QW), lambda i: (0, i, 0)),
        pl.BlockSpec((NQ, MB, QW), lambda i: (0, i, 0)),
        pl.BlockSpec((MB, 1), lambda i: (i, 0)),
        pl.BlockSpec((NQ, QW), lambda i: (0, 0)),
        pl.BlockSpec((NQ, QW), lambda i: (0, 0)),
        pl.BlockSpec((1, 1), lambda i: (0, 0)),
        pl.BlockSpec((1, 1), lambda i: (0, 0)),
        pl.BlockSpec((MB, 1), lambda i: (i, 0)),
    ],
    out_specs=pl.BlockSpec((MB, 1), lambda i: (i, 0)),
    out_shape=jax.ShapeDtypeStruct((N, 1), jnp.float32),
)


def kernel(x, edge_index, reranker_scores, W_in, b_in, W1, b1, W2, b2,
           W_out, b_out, alpha):
    ei = edge_index.astype(jnp.int32)
    src, dst = ei[0], ei[1]
    srcp = jnp.concatenate([src, jnp.zeros((EPAD - E,), jnp.int32)])
    dstp = jnp.concatenate([dst, jnp.full((EPAD - E,), N, jnp.int32)])
    src4 = (
        srcp[None, :] + (jnp.arange(NQ, dtype=jnp.int32) * N)[:, None]
    ).reshape(NQ, NT, NG, GC, CE)
    dst3h = dstp.reshape(NT, CPT, CHUNK)
    dst3 = dstp.reshape(NT, NG, GC, CE)

    dinv2d = _hist_dinv(dst3h)[:N, :1]

    hws1 = _tc_front(x, W_in, b_in.reshape(1, H), W1, dinv2d)
    agg1 = _agg(src4, dst3, hws1.reshape(NQ * N, QW)).reshape(NQ, N, QW)
    hws2 = _tc_mid(agg1, hws1, dinv2d, b1.reshape(NQ, 1, QW), W2)
    agg2 = _agg(src4, dst3, hws2.reshape(NQ * N, QW)).reshape(NQ, N, QW)
    out = _tc_final(
        agg2, hws2, dinv2d, b2.reshape(NQ, QW), W_out.reshape(NQ, QW),
        b_out.reshape(1, 1), alpha.reshape(1, 1), reranker_scores.reshape(N, 1),
    )
    return out.reshape(N)


# speedup vs baseline: 1.0293x; 1.0293x over previous
import functools

import jax
import jax.numpy as jnp
from jax import lax
from jax.experimental import pallas as pl
from jax.experimental.pallas import tpu as pltpu
from jax.experimental.pallas import tpu_sc as plsc

N = 10000
E = 160000
DIN = 256
H = 512
NQ = 4
QW = 128
NT = 16
CHUNK = 128
CPT = 79
CE = 64
NG = 2
GC = 79
EPAD = NT * CPT * CHUNK
NSL = 10240
ZPT = NSL // NT
DW = 128
MB = 1000

_SC_MESH = plsc.VectorSubcoreMesh(core_axis_name="c", subcore_axis_name="s")


def _zero_vec():
    return jnp.zeros((16,), jnp.float32)


@functools.partial(
    pl.kernel,
    out_type=jax.ShapeDtypeStruct((2, NSL, DW), jnp.float32),
    mesh=_SC_MESH,
    scratch_types=[
        pltpu.VMEM((CPT, CHUNK), jnp.int32),
        pltpu.VMEM((CHUNK, DW), jnp.float32),
        pltpu.VMEM((CHUNK, DW), jnp.float32),
        pltpu.VMEM_SHARED((NSL, DW), jnp.float32),
    ],
)
def _hist_part(dst_hbm, part_hbm, dbuf, ones_b, sbuf, hist_sh):
    c = lax.axis_index("c")
    s = lax.axis_index("s")
    one = jnp.ones((16,), jnp.float32)

    def fo(i, carry):
        for l in range(DW // 16):
            ones_b[i, pl.ds(l * 16, 16)] = one
            sbuf[i, pl.ds(l * 16, 16)] = _zero_vec()
        return carry

    lax.fori_loop(0, CHUNK, fo, 0)
    for k in range(ZPT // CHUNK):
        pltpu.sync_copy(sbuf, hist_sh.at[pl.ds(s * ZPT + k * CHUNK, CHUNK)])
    plsc.subcore_barrier()

    pltpu.sync_copy(dst_hbm.at[s], dbuf)

    def acc(i, carry):
        pltpu.sync_copy(ones_b, hist_sh.at[dbuf.at[i]], add=True)
        return carry

    lax.fori_loop(c * 40, 40 + c * (CPT - 40), acc, 0)
    plsc.subcore_barrier()

    for k in range(ZPT // CHUNK):
        pltpu.sync_copy(
            hist_sh.at[pl.ds(s * ZPT + k * CHUNK, CHUNK)],
            part_hbm.at[c, pl.ds(s * ZPT + k * CHUNK, CHUNK)],
        )


@functools.partial(
    pl.kernel,
    out_type=jax.ShapeDtypeStruct((NQ * N, QW), jnp.float32),
    mesh=_SC_MESH,
    scratch_types=[
        pltpu.VMEM((CPT, CHUNK), jnp.int32),
        pltpu.VMEM((CPT, CHUNK), jnp.int32),
        pltpu.VMEM((CHUNK, QW), jnp.float32),
        pltpu.VMEM_SHARED((NSL, QW), jnp.float32),
        pltpu.SemaphoreType.DMA,
    ],
)
def _agg(src_hbm, dst_hbm, hws_hbm, agg_hbm, isrc, idst, rows, slab, sem):
    c = lax.axis_index("c")
    s = lax.axis_index("s")
    rbase = s * ZPT

    for sub in range(2):
        q = c * 2 + sub

        def zr(i, carry):
            for l in range(QW // 16):
                rows[i, pl.ds(l * 16, 16)] = _zero_vec()
            return carry

        lax.fori_loop(0, CHUNK, zr, 0)
        for k in range(ZPT // CHUNK):
            pltpu.sync_copy(rows, slab.at[pl.ds(rbase + k * CHUNK, CHUNK)])
        plsc.subcore_barrier()

        pltpu.sync_copy(src_hbm.at[q, s], isrc)
        pltpu.sync_copy(dst_hbm.at[s], idst)

        def chunk(i, carry):
            pltpu.async_copy(hws_hbm.at[isrc.at[i]], rows, sem).wait()
            pltpu.sync_copy(rows, slab.at[idst.at[i]], add=True)
            return carry

        lax.fori_loop(0, CPT, chunk, 0)
        plsc.subcore_barrier()

        @pl.when(s < NT - 1)
        def _():
            pltpu.sync_copy(
                slab.at[pl.ds(rbase, ZPT)],
                agg_hbm.at[pl.ds(q * N + rbase, ZPT)],
            )

        @pl.when(s == NT - 1)
        def _():
            pltpu.sync_copy(
                slab.at[pl.ds(rbase, N - (NT - 1) * ZPT)],
                agg_hbm.at[pl.ds(q * N + rbase, N - (NT - 1) * ZPT)],
            )


def _front_body(x_ref, win_ref, bin_ref, w1_ref, part_ref, out_ref, dinv_ref):
    deg = part_ref[0, :, 0:1] + part_ref[1, :, 0:1] + 1.0
    dinv = lax.rsqrt(deg)
    dinv_ref[...] = dinv
    h0 = jnp.maximum(
        jnp.dot(x_ref[...], win_ref[...], preferred_element_type=jnp.float32)
        + bin_ref[...],
        0.0,
    )
    hw = jnp.dot(h0, w1_ref[...], preferred_element_type=jnp.float32)
    hws = hw * dinv
    for qq in range(NQ):
        out_ref[qq] = hws[:, qq * QW:(qq + 1) * QW]


_tc_front = pl.pallas_call(
    _front_body,
    grid=(N // MB,),
    in_specs=[
        pl.BlockSpec((MB, DIN), lambda i: (i, 0)),
        pl.BlockSpec((DIN, H), lambda i: (0, 0)),
        pl.BlockSpec((1, H), lambda i: (0, 0)),
        pl.BlockSpec((H, H), lambda i: (0, 0)),
        pl.BlockSpec((2, MB, DW), lambda i: (0, i, 0)),
    ],
    out_specs=[
        pl.BlockSpec((NQ, MB, QW), lambda i: (0, i, 0)),
        pl.BlockSpec((MB, 1), lambda i: (i, 0)),
    ],
    out_shape=[
        jax.ShapeDtypeStruct((NQ, N, QW), jnp.float32),
        jax.ShapeDtypeStruct((N, 1), jnp.float32),
    ],
)


def _mid_body(agg_ref, hws_ref, dinv_ref, b1_ref, w2_ref, out_ref):
    qi = pl.program_id(2)
    h = jnp.maximum(
        (agg_ref[0] + hws_ref[0]) * dinv_ref[...] + b1_ref[0], 0.0
    )
    p = jnp.dot(h, w2_ref[...], preferred_element_type=jnp.float32)

    @pl.when(qi == 0)
    def _():
        out_ref[0] = p

    @pl.when(qi > 0)
    def _():
        out_ref[0] = out_ref[0] + p

    @pl.when(qi == NQ - 1)
    def _():
        out_ref[0] = out_ref[0] * dinv_ref[...]


_tc_mid = pl.pallas_call(
    _mid_body,
    grid=(N // MB, NQ, NQ),
    in_specs=[
        pl.BlockSpec((1, MB, QW), lambda i, qo, qi: (qi, i, 0)),
        pl.BlockSpec((1, MB, QW), lambda i, qo, qi: (qi, i, 0)),
        pl.BlockSpec((MB, 1), lambda i, qo, qi: (i, 0)),
        pl.BlockSpec((1, 1, QW), lambda i, qo, qi: (qi, 0, 0)),
        pl.BlockSpec((QW, QW), lambda i, qo, qi: (qi, qo)),
    ],
    out_specs=pl.BlockSpec((1, MB, QW), lambda i, qo, qi: (qo, i, 0)),
    out_shape=jax.ShapeDtypeStruct((NQ, N, QW), jnp.float32),
)


def _final_body(agg_ref, hws_ref, dinv_ref, b2_ref, wout_ref, bout_ref,
                alpha_ref, rs_ref, out_ref):
    a = 1.0 / (1.0 + jnp.exp(-alpha_ref[0, 0]))
    dinv = dinv_ref[...]
    acc = jnp.zeros((MB, 1), jnp.float32)
    for qq in range(NQ):
        h2 = jnp.maximum(
            (agg_ref[qq] + hws_ref[qq]) * dinv + b2_ref[qq:qq + 1, :], 0.0
        )
        acc = acc + jnp.sum(h2 * wout_ref[qq:qq + 1, :], axis=1, keepdims=True)
    gnn = acc + bout_ref[0, 0]
    out_ref[...] = a * rs_ref[...] + (1.0 - a) * gnn


_tc_final = pl.pallas_call(
    _final_body,
    grid=(N // MB,),
    in_specs=[
        pl.BlockSpec((NQ, MB, QW), lambda i: (0, i, 0)),
        pl.BlockSpec((NQ, MB, QW), lambda i: (0, i, 0)),
        pl.BlockSpec((MB, 1), lambda i: (i, 0)),
        pl.BlockSpec((NQ, QW), lambda i: (0, 0)),
        pl.BlockSpec((NQ, QW), lambda i: (0, 0)),
        pl.BlockSpec((1, 1), lambda i: (0, 0)),
        pl.BlockSpec((1, 1), lambda i: (0, 0)),
        pl.BlockSpec((MB, 1), lambda i: (i, 0)),
    ],
    out_specs=pl.BlockSpec((MB, 1), lambda i: (i, 0)),
    out_shape=jax.ShapeDtypeStruct((N, 1), jnp.float32),
)


def kernel(x, edge_index, reranker_scores, W_in, b_in, W1, b1, W2, b2,
           W_out, b_out, alpha):
    ei = edge_index.astype(jnp.int32)
    src, dst = ei[0], ei[1]
    srcp = jnp.concatenate([src, jnp.zeros((EPAD - E,), jnp.int32)])
    dstp = jnp.concatenate([dst, jnp.full((EPAD - E,), N, jnp.int32)])
    src4 = (
        srcp[None, :] + (jnp.arange(NQ, dtype=jnp.int32) * N)[:, None]
    ).reshape(NQ, NT, CPT, CHUNK)
    dst3h = dstp.reshape(NT, CPT, CHUNK)
    dst3 = dstp.reshape(NT, CPT, CHUNK)

    parts = _hist_part(dst3h)

    hws1, dinv2d = _tc_front(x, W_in, b_in.reshape(1, H), W1, parts)
    agg1 = _agg(src4, dst3, hws1.reshape(NQ * N, QW)).reshape(NQ, N, QW)
    hws2 = _tc_mid(agg1, hws1, dinv2d, b1.reshape(NQ, 1, QW), W2)
    agg2 = _agg(src4, dst3, hws2.reshape(NQ * N, QW)).reshape(NQ, N, QW)
    out = _tc_final(
        agg2, hws2, dinv2d, b2.reshape(NQ, QW), W_out.reshape(NQ, QW),
        b_out.reshape(1, 1), alpha.reshape(1, 1), reranker_scores.reshape(N, 1),
    )
    return out.reshape(N)
